# Initial kernel scaffold; baseline (speedup 1.0000x reference)
import jax
import jax.numpy as jnp
from jax.experimental import pallas as pl


def kernel(x, token_table, pos_table):
    raise NotImplementedError("write your pallas kernel here")


# speedup vs baseline: 2.9058x; 2.9058x over previous
import functools

import jax
import jax.numpy as jnp
from jax import lax
from jax.experimental import pallas as pl
from jax.experimental.pallas import tpu as pltpu
from jax.experimental.pallas import tpu_sc as plsc

D = 128
LANES = 16
NC, NS = 2, 16
NW = NC * NS


def _embed_kernel(x_hbm, tok_hbm, pos_hbm, out_hbm,
                  idx_v, pos_v, row_v, sem):
    batch, n_chunks, _ = x_hbm.shape
    seq_len = n_chunks * D
    s_per_w = seq_len // NW
    c_per_w = s_per_w // D

    wid = lax.axis_index("s") * NC + lax.axis_index("c")
    s0 = wid * s_per_w

    pltpu.sync_copy(pos_hbm.at[pl.ds(s0, s_per_w)], pos_v)

    for b in range(batch):
        pltpu.sync_copy(x_hbm.at[b, pl.ds(wid * c_per_w, c_per_w)], idx_v)
        for h in range(c_per_w):
            pltpu.async_copy(tok_hbm.at[idx_v.at[h]],
                             row_v.at[pl.ds(h * D, D)], sem).wait()

        def add_row(r, _):
            for j in range(D // LANES):
                sl = pl.ds(j * LANES, LANES)
                row_v[r, sl] = row_v[r, sl] + pos_v[r, sl]
            return 0

        lax.fori_loop(0, s_per_w, add_row, 0)

        pltpu.sync_copy(row_v, out_hbm.at[b, pl.ds(s0, s_per_w)])


def kernel(x, token_table, pos_table):
    batch, seq_len = x.shape
    x3 = x.astype(jnp.int32).reshape(batch, seq_len // D, D)
    s_per_w = seq_len // NW

    mesh = plsc.VectorSubcoreMesh(core_axis_name="c", subcore_axis_name="s")
    run = pl.kernel(
        _embed_kernel,
        mesh=mesh,
        out_type=jax.ShapeDtypeStruct((batch, seq_len, D), jnp.float32),
        scratch_types=[
            pltpu.VMEM((seq_len // D // NW, D), jnp.int32),
            pltpu.VMEM((s_per_w, D), jnp.float32),
            pltpu.VMEM((s_per_w, D), jnp.float32),
            pltpu.SemaphoreType.DMA,
        ],
    )
    return run(x3, token_table, pos_table)


# speedup vs baseline: 3.0771x; 1.0590x over previous
import functools

import jax
import jax.numpy as jnp
from jax import lax
from jax.experimental import pallas as pl
from jax.experimental.pallas import tpu as pltpu
from jax.experimental.pallas import tpu_sc as plsc

D = 128
LANES = 16
NC, NS = 2, 16
NW = NC * NS


def _embed_kernel(x_hbm, tok_hbm, pos_hbm, out_hbm,
                  idx_v, row_v, sem):
    batch, n_chunks, _ = x_hbm.shape
    seq_len = n_chunks * D
    s_per_w = seq_len // NW
    c_per_w = s_per_w // D

    wid = lax.axis_index("s") * NC + lax.axis_index("c")
    s0 = wid * s_per_w

    for b in range(batch):
        pltpu.sync_copy(x_hbm.at[b, pl.ds(wid * c_per_w, c_per_w)], idx_v)
        pltpu.sync_copy(pos_hbm.at[pl.ds(s0, s_per_w)], row_v)
        for h in range(c_per_w):
            pltpu.async_copy(tok_hbm.at[idx_v.at[h]],
                             row_v.at[pl.ds(h * D, D)], sem, add=True).wait()

        pltpu.sync_copy(row_v, out_hbm.at[b, pl.ds(s0, s_per_w)])


def kernel(x, token_table, pos_table):
    batch, seq_len = x.shape
    x3 = x.astype(jnp.int32).reshape(batch, seq_len // D, D)
    s_per_w = seq_len // NW

    mesh = plsc.VectorSubcoreMesh(core_axis_name="c", subcore_axis_name="s")
    run = pl.kernel(
        _embed_kernel,
        mesh=mesh,
        out_type=jax.ShapeDtypeStruct((batch, seq_len, D), jnp.float32),
        scratch_types=[
            pltpu.VMEM((seq_len // D // NW, D), jnp.int32),
            pltpu.VMEM((s_per_w, D), jnp.float32),
            pltpu.SemaphoreType.DMA,
        ],
    )
    return run(x3, token_table, pos_table)


# speedup vs baseline: 3.7292x; 1.2119x over previous
import jax
import jax.numpy as jnp
from jax import lax
from jax.experimental import pallas as pl
from jax.experimental.pallas import tpu as pltpu
from jax.experimental.pallas import tpu_sc as plsc

D = 128
NC, NS = 2, 16
NW = NC * NS
NBUF = 4
ROWS = 128


def _embed_kernel(x_hbm, tok_hbm, pos_hbm, out_hbm,
                  idx_v, b0, b1, b2, b3,
                  isem, psems, gsems, ssems):
    batch, n_chunks, _ = x_hbm.shape
    seq_len = n_chunks * D
    s_per_w = seq_len // NW
    c_per_w = s_per_w // D
    n_tasks = batch * c_per_w

    wid = lax.axis_index("s") * NC + lax.axis_index("c")
    s0 = wid * s_per_w
    bufs = [b0, b1, b2, b3]

    def task_src(t):
        b, h = divmod(t, c_per_w)
        return b, s0 + h * ROWS

    hidx = pltpu.async_copy(
        x_hbm.at[pl.ds(0, batch), pl.ds(wid * c_per_w, c_per_w)],
        idx_v, isem)

    def prefill(t):
        _, s = task_src(t)
        return pltpu.async_copy(pos_hbm.at[pl.ds(s, ROWS)],
                                bufs[t % NBUF], psems.at[t % NBUF])

    hpre = {t: prefill(t) for t in range(NBUF)}
    hidx.wait()

    hg, hst = {}, {}

    def store(t):
        b, s = task_src(t)
        return pltpu.async_copy(bufs[t % NBUF],
                                out_hbm.at[b, pl.ds(s, ROWS)],
                                ssems.at[t % NBUF])

    for t in range(n_tasks):
        B = t % NBUF
        hpre[t].wait()
        b, h = divmod(t, c_per_w)
        hg[t] = pltpu.async_copy(tok_hbm.at[idx_v.at[b, h]],
                                 bufs[B], gsems.at[B], add=True)
        if t >= 1:
            hg[t - 1].wait()
            hst[t - 1] = store(t - 1)
        if t >= 2 and t + 2 < n_tasks:
            hst[t - 2].wait()
            hpre[t + 2] = prefill(t + 2)
    t = n_tasks - 1
    hg[t].wait()
    hst[t] = store(t)
    for t in range(n_tasks - 2, n_tasks):
        hst[t].wait()


def kernel(x, token_table, pos_table):
    batch, seq_len = x.shape
    x3 = x.astype(jnp.int32).reshape(batch, seq_len // D, D)

    mesh = plsc.VectorSubcoreMesh(core_axis_name="c", subcore_axis_name="s")
    run = pl.kernel(
        _embed_kernel,
        mesh=mesh,
        out_type=jax.ShapeDtypeStruct((batch, seq_len, D), jnp.float32),
        scratch_types=[
            pltpu.VMEM((batch, seq_len // D // NW, D), jnp.int32),
            pltpu.VMEM((ROWS, D), jnp.float32),
            pltpu.VMEM((ROWS, D), jnp.float32),
            pltpu.VMEM((ROWS, D), jnp.float32),
            pltpu.VMEM((ROWS, D), jnp.float32),
            pltpu.SemaphoreType.DMA,
            pltpu.SemaphoreType.DMA((NBUF,)),
            pltpu.SemaphoreType.DMA((NBUF,)),
            pltpu.SemaphoreType.DMA((NBUF,)),
        ],
    )
    return run(x3, token_table, pos_table)


# speedup vs baseline: 3.9796x; 1.0671x over previous
import jax
import jax.numpy as jnp
from jax import lax
from jax.experimental import pallas as pl
from jax.experimental.pallas import tpu as pltpu
from jax.experimental.pallas import tpu_sc as plsc

D = 128
NC, NS = 2, 16
NW = NC * NS
NBUF = 4
ROWS = 128


def _embed_kernel(x_hbm, tok_hbm, pos_hbm, out_hbm,
                  idx_v, b0, b1, b2, b3, pos_s,
                  isem, psems, gsems, ssems, stsem):
    batch, n_chunks, _ = x_hbm.shape
    seq_len = n_chunks * D
    s_per_w = seq_len // NW
    c_per_w = s_per_w // D
    n_tasks = batch * c_per_w

    sid = lax.axis_index("s")
    wid = sid * NC + lax.axis_index("c")
    s0 = wid * s_per_w
    p0 = sid * s_per_w
    bufs = [b0, b1, b2, b3]

    def task_src(t):
        b, h = divmod(t, c_per_w)
        return b, s0 + h * ROWS

    hidx = pltpu.async_copy(
        x_hbm.at[pl.ds(0, batch), pl.ds(wid * c_per_w, c_per_w)],
        idx_v, isem)

    pltpu.async_copy(pos_hbm.at[pl.ds(s0, s_per_w)],
                     pos_s.at[pl.ds(p0, s_per_w)], stsem).wait()

    def prefill(t):
        _, h = divmod(t, c_per_w)
        return pltpu.async_copy(pos_s.at[pl.ds(p0 + h * ROWS, ROWS)],
                                bufs[t % NBUF], psems.at[t % NBUF])

    hpre = {t: prefill(t) for t in range(NBUF)}
    hidx.wait()

    hg, hst = {}, {}

    def store(t):
        b, s = task_src(t)
        return pltpu.async_copy(bufs[t % NBUF],
                                out_hbm.at[b, pl.ds(s, ROWS)],
                                ssems.at[t % NBUF])

    for t in range(n_tasks):
        B = t % NBUF
        hpre[t].wait()
        b, h = divmod(t, c_per_w)
        hg[t] = pltpu.async_copy(tok_hbm.at[idx_v.at[b, h]],
                                 bufs[B], gsems.at[B], add=True)
        if t >= 1:
            hg[t - 1].wait()
            hst[t - 1] = store(t - 1)
        if t >= 2 and t + 2 < n_tasks:
            hst[t - 2].wait()
            hpre[t + 2] = prefill(t + 2)
    t = n_tasks - 1
    hg[t].wait()
    hst[t] = store(t)
    for t in range(n_tasks - 2, n_tasks):
        hst[t].wait()


def kernel(x, token_table, pos_table):
    batch, seq_len = x.shape
    x3 = x.astype(jnp.int32).reshape(batch, seq_len // D, D)

    mesh = plsc.VectorSubcoreMesh(core_axis_name="c", subcore_axis_name="s")
    run = pl.kernel(
        _embed_kernel,
        mesh=mesh,
        out_type=jax.ShapeDtypeStruct((batch, seq_len, D), jnp.float32),
        scratch_types=[
            pltpu.VMEM((batch, seq_len // D // NW, D), jnp.int32),
            pltpu.VMEM((ROWS, D), jnp.float32),
            pltpu.VMEM((ROWS, D), jnp.float32),
            pltpu.VMEM((ROWS, D), jnp.float32),
            pltpu.VMEM((ROWS, D), jnp.float32),
            pltpu.VMEM_SHARED((NS * (seq_len // NW), D),
                              jnp.float32),
            pltpu.SemaphoreType.DMA,
            pltpu.SemaphoreType.DMA((NBUF,)),
            pltpu.SemaphoreType.DMA((NBUF,)),
            pltpu.SemaphoreType.DMA((NBUF,)),
            pltpu.SemaphoreType.DMA,
        ],
    )
    return run(x3, token_table, pos_table)


# speedup vs baseline: 4.0087x; 1.0073x over previous
import jax
import jax.numpy as jnp
from jax import lax
from jax.experimental import pallas as pl
from jax.experimental.pallas import tpu as pltpu
from jax.experimental.pallas import tpu_sc as plsc

D = 128
NC, NS = 2, 16
NW = NC * NS
NBUF = 4
ROWS = 128


def _embed_kernel(x_hbm, tok_hbm, pos_hbm, out_hbm,
                  idx_v, b0, b1, b2, b3, pos_s,
                  isem, psems, gsems, ssems, stsem):
    batch, seq_len = x_hbm.shape
    s_per_w = seq_len // NW
    c_per_w = s_per_w // D
    n_tasks = batch * c_per_w

    sid = lax.axis_index("s")
    wid = sid * NC + lax.axis_index("c")
    s0 = wid * s_per_w
    p0 = sid * s_per_w
    bufs = [b0, b1, b2, b3]

    def task_src(t):
        b, h = divmod(t, c_per_w)
        return b, s0 + h * ROWS

    hidx = pltpu.async_copy(
        x_hbm.at[pl.ds(0, batch), pl.ds(s0, s_per_w)],
        idx_v, isem)

    pltpu.async_copy(pos_hbm.at[pl.ds(s0, s_per_w)],
                     pos_s.at[pl.ds(p0, s_per_w)], stsem).wait()

    def prefill(t):
        _, h = divmod(t, c_per_w)
        return pltpu.async_copy(pos_s.at[pl.ds(p0 + h * ROWS, ROWS)],
                                bufs[t % NBUF], psems.at[t % NBUF])

    hpre = {t: prefill(t) for t in range(NBUF)}
    hidx.wait()

    hg, hst = {}, {}

    def store(t):
        b, s = task_src(t)
        return pltpu.async_copy(bufs[t % NBUF],
                                out_hbm.at[b, pl.ds(s, ROWS)],
                                ssems.at[t % NBUF])

    for t in range(n_tasks):
        B = t % NBUF
        hpre[t].wait()
        b, h = divmod(t, c_per_w)
        hg[t] = pltpu.async_copy(tok_hbm.at[idx_v.at[b, pl.ds(h * ROWS, ROWS)]],
                                 bufs[B], gsems.at[B], add=True)
        if t >= 1:
            hg[t - 1].wait()
            hst[t - 1] = store(t - 1)
        if t >= 2 and t + 2 < n_tasks:
            hst[t - 2].wait()
            hpre[t + 2] = prefill(t + 2)
    t = n_tasks - 1
    hg[t].wait()
    hst[t] = store(t)
    for t in range(n_tasks - 2, n_tasks):
        hst[t].wait()


def kernel(x, token_table, pos_table):
    batch, seq_len = x.shape

    mesh = plsc.VectorSubcoreMesh(core_axis_name="c", subcore_axis_name="s")
    run = pl.kernel(
        _embed_kernel,
        mesh=mesh,
        out_type=jax.ShapeDtypeStruct((batch, seq_len, D), jnp.float32),
        scratch_types=[
            pltpu.VMEM((batch, seq_len // NW), jnp.int32),
            pltpu.VMEM((ROWS, D), jnp.float32),
            pltpu.VMEM((ROWS, D), jnp.float32),
            pltpu.VMEM((ROWS, D), jnp.float32),
            pltpu.VMEM((ROWS, D), jnp.float32),
            pltpu.VMEM_SHARED((NS * (seq_len // NW), D),
                              jnp.float32),
            pltpu.SemaphoreType.DMA,
            pltpu.SemaphoreType.DMA((NBUF,)),
            pltpu.SemaphoreType.DMA((NBUF,)),
            pltpu.SemaphoreType.DMA((NBUF,)),
            pltpu.SemaphoreType.DMA,
        ],
    )
    return run(x.astype(jnp.int32), token_table, pos_table)
